# Initial kernel scaffold; baseline (speedup 1.0000x reference)
import jax
import jax.numpy as jnp
from jax.experimental import pallas as pl


def kernel(mem, idx, val):
    raise NotImplementedError("write your pallas kernel here")


# speedup vs baseline: 1.9124x; 1.9124x over previous
import functools

import jax
import jax.numpy as jnp
from jax import lax
from jax.experimental import pallas as pl
from jax.experimental.pallas import tpu as pltpu
from jax.experimental.pallas import tpu_sc as plsc

NUM_NODES = 100000
MEMORY_DIM = 128
BATCH = 16384

_L = 16
_NC = 2
_NS = 16
_NW = _NC * _NS
_CHUNK = 2048
_B_PER_W = BATCH // _NW
_ROWS = 64
_JBITS = 14


def _body(idx_hbm, val_hbm, out_hbm, table_v, idx_v, w_v, rows_v, w_sh, sem):
    c = lax.axis_index("c")
    s = lax.axis_index("s")
    wid = s * _NC + c

    lanes = lax.iota(jnp.int32, _L)
    nxt_lane = jnp.minimum(lanes + 1, _L - 1)

    @pl.when(s == 0)
    def _phase1():
        def scatter_chunk(ci, _):
            pltpu.sync_copy(idx_hbm.at[pl.ds(ci * _CHUNK, _CHUNK)], idx_v)

            def scatter_vec(k, _):
                iv = idx_v[pl.ds(k * _L, _L)]
                jv = lanes + (ci * _CHUNK + k * _L)
                comb = iv * BATCH + jv
                comb_s, _ = plsc.sort_key_val(comb, comb)
                iv_s = lax.shift_right_arithmetic(comb_s, _JBITS)
                jv_s = jnp.bitwise_and(comb_s, BATCH - 1)
                nxt = iv_s.at[nxt_lane].get(mode="promise_in_bounds")
                keep = jnp.logical_or(iv_s != nxt, lanes == _L - 1)
                plsc.store_scatter(table_v, [iv_s], jv_s, mask=keep)
                return 0

            lax.fori_loop(0, _CHUNK // _L, scatter_vec, 0)
            return 0

        lax.fori_loop(0, BATCH // _CHUNK, scatter_chunk, 0)

        def wchunk(ci, _):
            pltpu.sync_copy(idx_hbm.at[pl.ds(ci * _CHUNK, _CHUNK)], idx_v)

            def wvec(k, _):
                iv = idx_v[pl.ds(k * _L, _L)]
                wv = plsc.load_gather(table_v, [iv])
                idx_v[pl.ds(k * _L, _L)] = wv
                return 0

            lax.fori_loop(0, _CHUNK // _L, wvec, 0)
            pltpu.sync_copy(idx_v, w_sh.at[pl.ds(ci * _CHUNK, _CHUNK)])
            return 0

        lax.fori_loop(0, BATCH // _CHUNK, wchunk, 0)

    plsc.subcore_barrier()

    base = wid * _B_PER_W
    pltpu.sync_copy(w_sh.at[pl.ds(base, _B_PER_W)], w_v)
    for t in range(_B_PER_W // _ROWS):
        pltpu.async_copy(
            val_hbm.at[w_v.at[pl.ds(t * _ROWS, _ROWS)]], rows_v, sem
        ).wait()
        pltpu.sync_copy(
            rows_v, out_hbm.at[pl.ds(base + t * _ROWS, _ROWS)]
        )


def kernel(mem, idx, val):
    del mem
    run = pl.kernel(
        _body,
        out_type=jax.ShapeDtypeStruct((BATCH, MEMORY_DIM), jnp.float32),
        mesh=plsc.VectorSubcoreMesh(core_axis_name="c", subcore_axis_name="s"),
        compiler_params=pltpu.CompilerParams(needs_layout_passes=False),
        scratch_types=[
            pltpu.VMEM((NUM_NODES,), jnp.int32),
            pltpu.VMEM((_CHUNK,), jnp.int32),
            pltpu.VMEM((_B_PER_W,), jnp.int32),
            pltpu.VMEM((_ROWS, MEMORY_DIM), jnp.float32),
            pltpu.VMEM_SHARED((BATCH,), jnp.int32),
            pltpu.SemaphoreType.DMA,
        ],
    )
    return run(idx, val)


# speedup vs baseline: 2.7259x; 1.4253x over previous
import functools

import jax
import jax.numpy as jnp
from jax import lax
from jax.experimental import pallas as pl
from jax.experimental.pallas import tpu as pltpu
from jax.experimental.pallas import tpu_sc as plsc

NUM_NODES = 100000
MEMORY_DIM = 128
BATCH = 16384

_L = 16
_NC = 2
_NS = 16
_NW = _NC * _NS
_B_PER_T = BATCH // _NS
_B_PER_W = BATCH // _NW
_ROWS = 32
_NBUF = 2
_JBITS = 14
_NPAD = 100352
_ZSLICE = _NPAD // _NS
_SENT = 0x7FFFFFFF


def _body(idx2d_hbm, val_hbm, out_hbm,
          table_v, idx_2d, rs_v, aux_v, dirty_v, w_v, cnt_v,
          rows_v, sum_sh, dirty_sh, cnt_sh, gsem, sems):
    c = lax.axis_index("c")
    s = lax.axis_index("s")
    wid = s * _NC + c
    lanes = lax.iota(jnp.int32, _L)
    nxt_lane = jnp.minimum(lanes + 1, _L - 1)
    jbase = s * _B_PER_T

    def zvec(k, _):
        aux_v[pl.ds(k * _L, _L)] = jnp.zeros((_L,), jnp.int32)
        return 0
    lax.fori_loop(0, _B_PER_T // _L, zvec, 0)
    zbase = s * _ZSLICE
    for r in range(_ZSLICE // _B_PER_T):
        pltpu.sync_copy(aux_v, sum_sh.at[pl.ds(zbase + r * _B_PER_T,
                                               _B_PER_T)])
    rem = _ZSLICE % _B_PER_T
    if rem:
        pltpu.sync_copy(aux_v.at[pl.ds(0, rem)],
                        sum_sh.at[pl.ds(zbase + _ZSLICE - rem, rem)])
    pltpu.sync_copy(idx2d_hbm.at[pl.ds(s * 8, 8)], idx_2d)
    plsc.subcore_barrier()

    for r in range(8):
        def ramp(k2, _):
            j0 = jbase + r * 128 + k2 * _L
            rs_v.at[r][pl.ds(k2 * _L, _L)] = (lanes + j0) * BATCH + 1
            return 0
        lax.fori_loop(0, 8, ramp, 0)
    cps = [
        pltpu.async_copy(rs_v.at[r], sum_sh.at[idx_2d.at[r]], sems.at[r % 2],
                         add=True)
        for r in range(8)
    ]
    for cp in cps:
        cp.wait()
    plsc.subcore_barrier()

    gps = [
        pltpu.async_copy(sum_sh.at[idx_2d.at[r]], rs_v.at[r], sems.at[r % 2])
        for r in range(8)
    ]
    for cp in gps:
        cp.wait()

    ndirty = jnp.int32(0)
    for r in range(8):
        def compact(k2, off):
            iv = idx_2d.at[r][pl.ds(k2 * _L, _L)]
            sv = rs_v.at[r][pl.ds(k2 * _L, _L)]
            jv = lanes + (jbase + r * 128 + k2 * _L)
            dirtym = sv != jv * BATCH + 1
            comb = iv * BATCH + jv
            plsc.store_compressed(dirty_v.at[pl.ds(off, _L)], comb,
                                  mask=dirtym)
            return off + jnp.sum(dirtym.astype(jnp.int32))
        ndirty = lax.fori_loop(0, 8, compact, ndirty)

    pltpu.sync_copy(dirty_v, dirty_sh.at[s])
    aux_v[pl.ds(0, _L)] = jnp.broadcast_to(ndirty, (_L,)).astype(jnp.int32)
    pltpu.sync_copy(aux_v.at[pl.ds(0, _L)], cnt_sh.at[s])
    plsc.subcore_barrier()

    @pl.when(s == 0)
    def _fixup():
        pltpu.sync_copy(cnt_sh, cnt_v)
        for t in range(_NS):
            ct = jnp.max(cnt_v.at[t][...])
            pltpu.sync_copy(dirty_sh.at[t], aux_v)

            def replay(v, _):
                cm = aux_v[pl.ds(v * _L, _L)]
                maskv = lanes < ct - v * _L
                cmw = jnp.where(maskv, cm, _SENT)
                cs, _ = plsc.sort_key_val(cmw, cmw)
                ivs = lax.shift_right_arithmetic(cs, _JBITS)
                jvs = jnp.bitwise_and(cs, BATCH - 1)
                nxt = ivs.at[nxt_lane].get(mode="promise_in_bounds")
                keep = jnp.logical_or(ivs != nxt, lanes == _L - 1)
                keep = jnp.logical_and(keep, maskv)
                plsc.store_scatter(table_v, [ivs], jvs, mask=keep)
                return 0

            lax.fori_loop(0, (ct + _L - 1) // _L, replay, 0)
        for t in range(_NS):
            ct = jnp.max(cnt_v.at[t][...])
            pltpu.sync_copy(dirty_sh.at[t], aux_v)

            def winners(v, _):
                cm = aux_v[pl.ds(v * _L, _L)]
                maskv = lanes < ct - v * _L
                iv = lax.shift_right_arithmetic(
                    jnp.where(maskv, cm, 0), _JBITS)
                wv = plsc.load_gather(table_v, [iv], mask=maskv)
                aux_v[pl.ds(v * _L, _L)] = jnp.where(maskv, wv, cm)
                return 0

            lax.fori_loop(0, (ct + _L - 1) // _L, winners, 0)
            pltpu.sync_copy(aux_v, dirty_sh.at[t])
    plsc.subcore_barrier()

    def wid_init(k, _):
        w_v[pl.ds(k * _L, _L)] = lanes + (jbase + k * _L)
        return 0
    lax.fori_loop(0, _B_PER_T // _L, wid_init, 0)

    pltpu.sync_copy(dirty_sh.at[s], aux_v)

    def apply_fix(v, _):
        cm = dirty_v[pl.ds(v * _L, _L)]
        wv = aux_v[pl.ds(v * _L, _L)]
        maskv = lanes < ndirty - v * _L
        jloc = jnp.bitwise_and(cm, BATCH - 1) - jbase
        plsc.store_scatter(w_v, [jloc], wv, mask=maskv)
        return 0

    lax.fori_loop(0, (ndirty + _L - 1) // _L, apply_fix, 0)

    base = wid * _B_PER_W
    woff = c * _B_PER_W
    nch = _B_PER_W // _ROWS

    def gather_cp(t, buf):
        return pltpu.async_copy(
            val_hbm.at[w_v.at[pl.ds(woff + t * _ROWS, _ROWS)]],
            rows_v.at[buf], gsem.at[buf])

    cp0 = gather_cp(0, 0)
    prev = cp0
    for t in range(nch):
        nxtcp = gather_cp(t + 1, (t + 1) % _NBUF) if t + 1 < nch else None
        prev.wait()
        pltpu.sync_copy(rows_v.at[t % _NBUF],
                        out_hbm.at[pl.ds(base + t * _ROWS, _ROWS)])
        prev = nxtcp


def kernel(mem, idx, val):
    del mem
    run = pl.kernel(
        _body,
        out_type=jax.ShapeDtypeStruct((BATCH, MEMORY_DIM), jnp.float32),
        mesh=plsc.VectorSubcoreMesh(core_axis_name="c", subcore_axis_name="s"),
        compiler_params=pltpu.CompilerParams(needs_layout_passes=False),
        scratch_types=[
            pltpu.VMEM((NUM_NODES,), jnp.int32),
            pltpu.VMEM((8, 128), jnp.int32),
            pltpu.VMEM((8, 128), jnp.int32),
            pltpu.VMEM((_B_PER_T,), jnp.int32),
            pltpu.VMEM((_B_PER_T,), jnp.int32),
            pltpu.VMEM((_B_PER_T,), jnp.int32),
            pltpu.VMEM((_NS, _L), jnp.int32),
            pltpu.VMEM((_NBUF, _ROWS, MEMORY_DIM), jnp.float32),
            pltpu.VMEM_SHARED((_NPAD,), jnp.int32),
            pltpu.VMEM_SHARED((_NS, _B_PER_T), jnp.int32),
            pltpu.VMEM_SHARED((_NS, _L), jnp.int32),
            pltpu.SemaphoreType.DMA((_NBUF,)),
            pltpu.SemaphoreType.DMA((2,)),
        ],
    )
    return run(idx.reshape(BATCH // 128, 128), val)


# speedup vs baseline: 2.9777x; 1.0924x over previous
import functools

import jax
import jax.numpy as jnp
from jax import lax
from jax.experimental import pallas as pl
from jax.experimental.pallas import tpu as pltpu
from jax.experimental.pallas import tpu_sc as plsc

NUM_NODES = 100000
MEMORY_DIM = 128
BATCH = 16384

_L = 16
_NC = 2
_NS = 16
_NW = _NC * _NS
_B_PER_T = BATCH // _NS
_B_PER_W = BATCH // _NW
_ROWS = 32
_NBUF = 2
_JBITS = 14
_NPAD = 100352
_ZSLICE = _NPAD // _NS
_SENT = 0x7FFFFFFF


def _body(idx2d_hbm, val_hbm, out_hbm,
          table_v, idx_2d, rs_v, aux_v, fixb_v, dirty_v, w_v, cnt_v,
          rows_v, sum_sh, dirty_sh, cnt_sh, gsem, sems, bsem, wsem):
    c = lax.axis_index("c")
    s = lax.axis_index("s")
    wid = s * _NC + c
    lanes = lax.iota(jnp.int32, _L)
    nxt_lane = jnp.minimum(lanes + 1, _L - 1)
    jbase = s * _B_PER_T

    def zvec(k, _):
        aux_v[pl.ds(k * _L, _L)] = jnp.zeros((_L,), jnp.int32)
        return 0
    lax.fori_loop(0, _B_PER_T // _L, zvec, 0)
    zbase = s * _ZSLICE
    zcps = [
        pltpu.async_copy(aux_v, sum_sh.at[pl.ds(zbase + r * _B_PER_T,
                                                _B_PER_T)], sems.at[r % 2])
        for r in range(_ZSLICE // _B_PER_T)
    ]
    rem = _ZSLICE % _B_PER_T
    zcps.append(pltpu.async_copy(
        aux_v.at[pl.ds(0, rem)],
        sum_sh.at[pl.ds(zbase + _ZSLICE - rem, rem)], sems.at[0]))
    pltpu.sync_copy(idx2d_hbm.at[pl.ds(s * 8, 8)], idx_2d)
    for r in range(8):
        def ramp(k2, _):
            j0 = jbase + r * 128 + k2 * _L
            rs_v.at[r][pl.ds(k2 * _L, _L)] = (lanes + j0) * BATCH + 1
            return 0
        lax.fori_loop(0, 8, ramp, 0)
    for cp in zcps:
        cp.wait()
    plsc.subcore_barrier()

    cps = [
        pltpu.async_copy(rs_v.at[r], sum_sh.at[idx_2d.at[r]], sems.at[r % 2],
                         add=True)
        for r in range(8)
    ]
    for cp in cps:
        cp.wait()
    plsc.subcore_barrier()

    gps = [
        pltpu.async_copy(sum_sh.at[idx_2d.at[r]], rs_v.at[r], sems.at[r % 2])
        for r in range(8)
    ]
    for cp in gps:
        cp.wait()

    ndirty = jnp.int32(0)
    for r in range(8):
        def compact(k2, off):
            iv = idx_2d.at[r][pl.ds(k2 * _L, _L)]
            sv = rs_v.at[r][pl.ds(k2 * _L, _L)]
            jv = lanes + (jbase + r * 128 + k2 * _L)
            dirtym = sv != jv * BATCH + 1
            comb = iv * BATCH + jv
            plsc.store_compressed(dirty_v.at[pl.ds(off, _L)], comb,
                                  mask=dirtym)
            return off + jnp.sum(dirtym.astype(jnp.int32))
        ndirty = lax.fori_loop(0, 8, compact, ndirty)

    pltpu.sync_copy(dirty_v, dirty_sh.at[s])
    aux_v[pl.ds(0, _L)] = jnp.broadcast_to(ndirty, (_L,)).astype(jnp.int32)
    pltpu.sync_copy(aux_v.at[pl.ds(0, _L)], cnt_sh.at[s])
    plsc.subcore_barrier()

    @pl.when(s == 0)
    def _fixup():
        pltpu.sync_copy(cnt_sh, cnt_v)
        bufs = [aux_v, fixb_v]
        cp = pltpu.async_copy(dirty_sh.at[0], bufs[0], bsem.at[0])
        for t in range(_NS):
            ct = jnp.max(cnt_v.at[t][...])
            cpn = (pltpu.async_copy(dirty_sh.at[t + 1], bufs[(t + 1) % 2],
                                    bsem.at[(t + 1) % 2])
                   if t + 1 < _NS else None)
            cp.wait()
            buf = bufs[t % 2]

            def replay(v, _):
                cm = buf[pl.ds(v * _L, _L)]
                maskv = lanes < ct - v * _L
                cmw = jnp.where(maskv, cm, _SENT)
                cs, _ = plsc.sort_key_val(cmw, cmw)
                ivs = lax.shift_right_arithmetic(cs, _JBITS)
                jvs = jnp.bitwise_and(cs, BATCH - 1)
                nxt = ivs.at[nxt_lane].get(mode="promise_in_bounds")
                keep = jnp.logical_or(ivs != nxt, lanes == _L - 1)
                keep = jnp.logical_and(keep, maskv)
                plsc.store_scatter(table_v, [ivs], jvs, mask=keep)
                return 0

            lax.fori_loop(0, (ct + _L - 1) // _L, replay, 0)
            cp = cpn
        cp = pltpu.async_copy(dirty_sh.at[0], bufs[0], bsem.at[0])
        wbs = [None, None]
        for t in range(_NS):
            ct = jnp.max(cnt_v.at[t][...])
            cpn = None
            if t + 1 < _NS:
                if wbs[(t + 1) % 2] is not None:
                    wbs[(t + 1) % 2].wait()
                    wbs[(t + 1) % 2] = None
                cpn = pltpu.async_copy(dirty_sh.at[t + 1], bufs[(t + 1) % 2],
                                       bsem.at[(t + 1) % 2])
            cp.wait()
            buf = bufs[t % 2]

            def winners(v, _):
                cm = buf[pl.ds(v * _L, _L)]
                maskv = lanes < ct - v * _L
                iv = lax.shift_right_arithmetic(
                    jnp.where(maskv, cm, 0), _JBITS)
                wv = plsc.load_gather(table_v, [iv], mask=maskv)
                buf[pl.ds(v * _L, _L)] = jnp.where(maskv, wv, cm)
                return 0

            lax.fori_loop(0, (ct + _L - 1) // _L, winners, 0)
            wbs[t % 2] = pltpu.async_copy(buf, dirty_sh.at[t],
                                          wsem.at[t % 2])
            cp = cpn
        for wb in wbs:
            if wb is not None:
                wb.wait()
    plsc.subcore_barrier()

    def wid_init(k, _):
        w_v[pl.ds(k * _L, _L)] = lanes + (jbase + k * _L)
        return 0
    lax.fori_loop(0, _B_PER_T // _L, wid_init, 0)

    pltpu.sync_copy(dirty_sh.at[s], aux_v)

    def apply_fix(v, _):
        cm = dirty_v[pl.ds(v * _L, _L)]
        wv = aux_v[pl.ds(v * _L, _L)]
        maskv = lanes < ndirty - v * _L
        jloc = jnp.bitwise_and(cm, BATCH - 1) - jbase
        plsc.store_scatter(w_v, [jloc], wv, mask=maskv)
        return 0

    lax.fori_loop(0, (ndirty + _L - 1) // _L, apply_fix, 0)

    base = wid * _B_PER_W
    woff = c * _B_PER_W
    nch = _B_PER_W // _ROWS

    def gather_cp(t, buf):
        return pltpu.async_copy(
            val_hbm.at[w_v.at[pl.ds(woff + t * _ROWS, _ROWS)]],
            rows_v.at[buf], gsem.at[buf])

    gcur = gather_cp(0, 0)
    for t in range(nch):
        gnxt = gather_cp(t + 1, (t + 1) % _NBUF) if t + 1 < nch else None
        gcur.wait()
        pltpu.sync_copy(rows_v.at[t % _NBUF],
                        out_hbm.at[pl.ds(base + t * _ROWS, _ROWS)])
        gcur = gnxt


def kernel(mem, idx, val):
    del mem
    run = pl.kernel(
        _body,
        out_type=jax.ShapeDtypeStruct((BATCH, MEMORY_DIM), jnp.float32),
        mesh=plsc.VectorSubcoreMesh(core_axis_name="c", subcore_axis_name="s"),
        compiler_params=pltpu.CompilerParams(needs_layout_passes=False),
        scratch_types=[
            pltpu.VMEM((NUM_NODES,), jnp.int32),
            pltpu.VMEM((8, 128), jnp.int32),
            pltpu.VMEM((8, 128), jnp.int32),
            pltpu.VMEM((_B_PER_T,), jnp.int32),
            pltpu.VMEM((_B_PER_T,), jnp.int32),
            pltpu.VMEM((_B_PER_T,), jnp.int32),
            pltpu.VMEM((_B_PER_T,), jnp.int32),
            pltpu.VMEM((_NS, _L), jnp.int32),
            pltpu.VMEM((_NBUF, _ROWS, MEMORY_DIM), jnp.float32),
            pltpu.VMEM_SHARED((_NPAD,), jnp.int32),
            pltpu.VMEM_SHARED((_NS, _B_PER_T), jnp.int32),
            pltpu.VMEM_SHARED((_NS, _L), jnp.int32),
            pltpu.SemaphoreType.DMA((_NBUF,)),
            pltpu.SemaphoreType.DMA((2,)),
            pltpu.SemaphoreType.DMA((2,)),
            pltpu.SemaphoreType.DMA((2,)),
        ],
    )
    return run(idx.reshape(BATCH // 128, 128), val)


# speedup vs baseline: 3.0323x; 1.0184x over previous
import functools

import jax
import jax.numpy as jnp
from jax import lax
from jax.experimental import pallas as pl
from jax.experimental.pallas import tpu as pltpu
from jax.experimental.pallas import tpu_sc as plsc

NUM_NODES = 100000
MEMORY_DIM = 128
BATCH = 16384

_L = 16
_NC = 2
_NS = 16
_NW = _NC * _NS
_B_PER_T = BATCH // _NS
_B_PER_W = BATCH // _NW
_ROWS = 32
_NBUF = 2
_JBITS = 14
_NPAD = 100352
_ZSLICE = _NPAD // _NS
_SENT = 0x7FFFFFFF


def _body(idx2d_hbm, val_hbm, out_hbm,
          table_v, idx_2d, rs_v, aux_v, fixb_v, dirty_v, w_v, cnt_v,
          rows_v, sum_sh, dirty_sh, cnt_sh, gsem, sems, bsem, wsem):
    c = lax.axis_index("c")
    s = lax.axis_index("s")
    wid = s * _NC + c
    lanes = lax.iota(jnp.int32, _L)
    nxt_lane = jnp.minimum(lanes + 1, _L - 1)
    jbase = s * _B_PER_T

    def zvec(k, _):
        aux_v[pl.ds(k * _L, _L)] = jnp.zeros((_L,), jnp.int32)
        return 0
    lax.fori_loop(0, _B_PER_T // _L, zvec, 0)
    zbase = s * _ZSLICE
    zcps = [
        pltpu.async_copy(aux_v, sum_sh.at[pl.ds(zbase + r * _B_PER_T,
                                                _B_PER_T)], sems.at[r % 2])
        for r in range(_ZSLICE // _B_PER_T)
    ]
    rem = _ZSLICE % _B_PER_T
    zcps.append(pltpu.async_copy(
        aux_v.at[pl.ds(0, rem)],
        sum_sh.at[pl.ds(zbase + _ZSLICE - rem, rem)], sems.at[0]))
    pltpu.sync_copy(idx2d_hbm.at[pl.ds(s * 8, 8)], idx_2d)
    for r in range(8):
        def ramp(k2, _):
            j0 = jbase + r * 128 + k2 * _L
            rs_v.at[r][pl.ds(k2 * _L, _L)] = (lanes + j0) * BATCH + 1
            return 0
        lax.fori_loop(0, 8, ramp, 0)
    for cp in zcps:
        cp.wait()
    plsc.subcore_barrier()

    cps = [
        pltpu.async_copy(rs_v.at[r], sum_sh.at[idx_2d.at[r]], sems.at[r % 2],
                         add=True)
        for r in range(8)
    ]
    for cp in cps:
        cp.wait()
    plsc.subcore_barrier()

    gps = [
        pltpu.async_copy(sum_sh.at[idx_2d.at[r]], rs_v.at[r], sems.at[r % 2])
        for r in range(8)
    ]
    for cp in gps:
        cp.wait()

    ndirty = jnp.int32(0)
    for r in range(8):
        def compact(k2, off):
            iv = idx_2d.at[r][pl.ds(k2 * _L, _L)]
            sv = rs_v.at[r][pl.ds(k2 * _L, _L)]
            jv = lanes + (jbase + r * 128 + k2 * _L)
            dirtym = sv != jv * BATCH + 1
            comb = iv * BATCH + jv
            plsc.store_compressed(dirty_v.at[pl.ds(off, _L)], comb,
                                  mask=dirtym)
            return off + jnp.sum(dirtym.astype(jnp.int32))
        ndirty = lax.fori_loop(0, 8, compact, ndirty)

    pltpu.sync_copy(dirty_v, dirty_sh.at[s])
    aux_v[pl.ds(0, _L)] = jnp.broadcast_to(ndirty, (_L,)).astype(jnp.int32)
    pltpu.sync_copy(aux_v.at[pl.ds(0, _L)], cnt_sh.at[s])
    plsc.subcore_barrier()

    @pl.when(s == 0)
    def _fixup():
        pltpu.sync_copy(cnt_sh, cnt_v)
        bufs = [aux_v, fixb_v]
        cp = pltpu.async_copy(dirty_sh.at[0], bufs[0], bsem.at[0])
        for t in range(_NS):
            ct = jnp.max(cnt_v.at[t][...])
            cpn = (pltpu.async_copy(dirty_sh.at[t + 1], bufs[(t + 1) % 2],
                                    bsem.at[(t + 1) % 2])
                   if t + 1 < _NS else None)
            cp.wait()
            buf = bufs[t % 2]

            def replay(v, _):
                cm = buf[pl.ds(v * _L, _L)]
                maskv = lanes < ct - v * _L
                cmw = jnp.where(maskv, cm, _SENT)
                cs, _ = plsc.sort_key_val(cmw, cmw)
                ivs = lax.shift_right_arithmetic(cs, _JBITS)
                jvs = jnp.bitwise_and(cs, BATCH - 1)
                nxt = ivs.at[nxt_lane].get(mode="promise_in_bounds")
                keep = jnp.logical_or(ivs != nxt, lanes == _L - 1)
                keep = jnp.logical_and(keep, maskv)
                plsc.store_scatter(table_v, [ivs], jvs, mask=keep)
                return 0

            lax.fori_loop(0, (ct + _L - 1) // _L, replay, 0)
            cp = cpn
        cp = pltpu.async_copy(dirty_sh.at[0], bufs[0], bsem.at[0])
        wbs = [None, None]
        for t in range(_NS):
            ct = jnp.max(cnt_v.at[t][...])
            cpn = None
            if t + 1 < _NS:
                if wbs[(t + 1) % 2] is not None:
                    wbs[(t + 1) % 2].wait()
                    wbs[(t + 1) % 2] = None
                cpn = pltpu.async_copy(dirty_sh.at[t + 1], bufs[(t + 1) % 2],
                                       bsem.at[(t + 1) % 2])
            cp.wait()
            buf = bufs[t % 2]

            def winners(v, _):
                cm = buf[pl.ds(v * _L, _L)]
                maskv = lanes < ct - v * _L
                iv = lax.shift_right_arithmetic(
                    jnp.where(maskv, cm, 0), _JBITS)
                wv = plsc.load_gather(table_v, [iv], mask=maskv)
                buf[pl.ds(v * _L, _L)] = jnp.where(maskv, wv, cm)
                return 0

            lax.fori_loop(0, (ct + _L - 1) // _L, winners, 0)
            wbs[t % 2] = pltpu.async_copy(buf, dirty_sh.at[t],
                                          wsem.at[t % 2])
            cp = cpn
        for wb in wbs:
            if wb is not None:
                wb.wait()
    plsc.subcore_barrier()

    fixcp = pltpu.async_copy(dirty_sh.at[s], aux_v, bsem.at[0])

    def wid_init(k, _):
        w_v[pl.ds(k * _L, _L)] = lanes + (jbase + k * _L)
        return 0
    lax.fori_loop(0, _B_PER_T // _L, wid_init, 0)
    fixcp.wait()

    def apply_fix(v, _):
        cm = dirty_v[pl.ds(v * _L, _L)]
        wv = aux_v[pl.ds(v * _L, _L)]
        maskv = lanes < ndirty - v * _L
        jloc = jnp.bitwise_and(cm, BATCH - 1) - jbase
        plsc.store_scatter(w_v, [jloc], wv, mask=maskv)
        return 0

    lax.fori_loop(0, (ndirty + _L - 1) // _L, apply_fix, 0)

    base = wid * _B_PER_W
    woff = c * _B_PER_W
    nch = _B_PER_W // _ROWS

    def gather_cp(t, buf):
        return pltpu.async_copy(
            val_hbm.at[w_v.at[pl.ds(woff + t * _ROWS, _ROWS)]],
            rows_v.at[buf], gsem.at[buf])

    gcur = gather_cp(0, 0)
    sts = [None, None]
    for t in range(nch):
        gnxt = None
        if t + 1 < nch:
            b = (t + 1) % _NBUF
            if sts[b] is not None:
                sts[b].wait()
                sts[b] = None
            gnxt = gather_cp(t + 1, b)
        gcur.wait()
        sts[t % _NBUF] = pltpu.async_copy(
            rows_v.at[t % _NBUF],
            out_hbm.at[pl.ds(base + t * _ROWS, _ROWS)],
            wsem.at[t % _NBUF])
        gcur = gnxt
    for st in sts:
        if st is not None:
            st.wait()


def kernel(mem, idx, val):
    del mem
    run = pl.kernel(
        _body,
        out_type=jax.ShapeDtypeStruct((BATCH, MEMORY_DIM), jnp.float32),
        mesh=plsc.VectorSubcoreMesh(core_axis_name="c", subcore_axis_name="s"),
        compiler_params=pltpu.CompilerParams(needs_layout_passes=False),
        scratch_types=[
            pltpu.VMEM((NUM_NODES,), jnp.int32),
            pltpu.VMEM((8, 128), jnp.int32),
            pltpu.VMEM((8, 128), jnp.int32),
            pltpu.VMEM((_B_PER_T,), jnp.int32),
            pltpu.VMEM((_B_PER_T,), jnp.int32),
            pltpu.VMEM((_B_PER_T,), jnp.int32),
            pltpu.VMEM((_B_PER_T,), jnp.int32),
            pltpu.VMEM((_NS, _L), jnp.int32),
            pltpu.VMEM((_NBUF, _ROWS, MEMORY_DIM), jnp.float32),
            pltpu.VMEM_SHARED((_NPAD,), jnp.int32),
            pltpu.VMEM_SHARED((_NS, _B_PER_T), jnp.int32),
            pltpu.VMEM_SHARED((_NS, _L), jnp.int32),
            pltpu.SemaphoreType.DMA((_NBUF,)),
            pltpu.SemaphoreType.DMA((2,)),
            pltpu.SemaphoreType.DMA((2,)),
            pltpu.SemaphoreType.DMA((2,)),
        ],
    )
    return run(idx.reshape(BATCH // 128, 128), val)


# speedup vs baseline: 3.0591x; 1.0088x over previous
import functools

import jax
import jax.numpy as jnp
from jax import lax
from jax.experimental import pallas as pl
from jax.experimental.pallas import tpu as pltpu
from jax.experimental.pallas import tpu_sc as plsc

NUM_NODES = 100000
MEMORY_DIM = 128
BATCH = 16384

_L = 16
_NC = 2
_NS = 16
_NW = _NC * _NS
_B_PER_T = BATCH // _NS
_B_PER_W = BATCH // _NW
_ROWS = 32
_NBUF = 2
_JBITS = 14
_NPAD = 100352
_ZSLICE = _NPAD // _NS
_SENT = 0x7FFFFFFF


def _body(idx2d_hbm, val_hbm, out_hbm,
          table_v, idx_2d, rs_v, aux_v, fixb_v, dirty_v, w_v, cnt_v,
          rows_v, sum_sh, dirty_sh, cnt_sh, gsem, sems, bsem, wsem):
    c = lax.axis_index("c")
    s = lax.axis_index("s")
    wid = s * _NC + c
    lanes = lax.iota(jnp.int32, _L)
    nxt_lane = jnp.minimum(lanes + 1, _L - 1)
    jbase = s * _B_PER_T

    def zvec(k, _):
        aux_v[pl.ds(k * _L, _L)] = jnp.zeros((_L,), jnp.int32)
        return 0
    lax.fori_loop(0, _B_PER_T // _L, zvec, 0)
    zbase = s * _ZSLICE
    zcps = [
        pltpu.async_copy(aux_v, sum_sh.at[pl.ds(zbase + r * _B_PER_T,
                                                _B_PER_T)], sems.at[r % 2])
        for r in range(_ZSLICE // _B_PER_T)
    ]
    rem = _ZSLICE % _B_PER_T
    zcps.append(pltpu.async_copy(
        aux_v.at[pl.ds(0, rem)],
        sum_sh.at[pl.ds(zbase + _ZSLICE - rem, rem)], sems.at[0]))
    pltpu.sync_copy(idx2d_hbm.at[pl.ds(s * 8, 8)], idx_2d)
    for r in range(8):
        def ramp(k2, _):
            j0 = jbase + r * 128 + k2 * _L
            rs_v.at[r][pl.ds(k2 * _L, _L)] = (lanes + j0) * BATCH + 1
            return 0
        lax.fori_loop(0, 8, ramp, 0)
    for cp in zcps:
        cp.wait()
    plsc.subcore_barrier()

    cps = [
        pltpu.async_copy(rs_v.at[r], sum_sh.at[idx_2d.at[r]], sems.at[r % 2],
                         add=True)
        for r in range(8)
    ]
    for cp in cps:
        cp.wait()
    plsc.subcore_barrier()

    gps = [
        pltpu.async_copy(sum_sh.at[idx_2d.at[r]], rs_v.at[r], sems.at[r % 2])
        for r in range(8)
    ]
    for cp in gps:
        cp.wait()

    ndirty = jnp.int32(0)
    for r in range(8):
        def compact(k2, off):
            iv = idx_2d.at[r][pl.ds(k2 * _L, _L)]
            sv = rs_v.at[r][pl.ds(k2 * _L, _L)]
            jv = lanes + (jbase + r * 128 + k2 * _L)
            dirtym = sv != jv * BATCH + 1
            comb = iv * BATCH + jv
            plsc.store_compressed(dirty_v.at[pl.ds(off, _L)], comb,
                                  mask=dirtym)
            return off + jnp.sum(dirtym.astype(jnp.int32))
        ndirty = lax.fori_loop(0, 8, compact, ndirty)

    pltpu.sync_copy(dirty_v, dirty_sh.at[s])
    aux_v[pl.ds(0, _L)] = jnp.broadcast_to(ndirty, (_L,)).astype(jnp.int32)
    pltpu.sync_copy(aux_v.at[pl.ds(0, _L)], cnt_sh.at[s])
    plsc.subcore_barrier()

    @pl.when(s == 0)
    def _fixup():
        pltpu.sync_copy(cnt_sh, cnt_v)
        bufs = [aux_v, fixb_v]
        cp = pltpu.async_copy(dirty_sh.at[0], bufs[0], bsem.at[0])
        for t in range(_NS):
            ct = jnp.max(cnt_v.at[t][...])
            cpn = (pltpu.async_copy(dirty_sh.at[t + 1], bufs[(t + 1) % 2],
                                    bsem.at[(t + 1) % 2])
                   if t + 1 < _NS else None)
            cp.wait()
            buf = bufs[t % 2]

            def replay(v, _):
                cm = buf[pl.ds(v * _L, _L)]
                maskv = lanes < ct - v * _L
                cmw = jnp.where(maskv, cm, _SENT)
                cs, _ = plsc.sort_key_val(cmw, cmw)
                ivs = lax.shift_right_arithmetic(cs, _JBITS)
                jvs = jnp.bitwise_and(cs, BATCH - 1)
                nxt = ivs.at[nxt_lane].get(mode="promise_in_bounds")
                keep = jnp.logical_or(ivs != nxt, lanes == _L - 1)
                keep = jnp.logical_and(keep, maskv)
                plsc.store_scatter(table_v, [ivs], jvs, mask=keep)
                return 0

            lax.fori_loop(0, (ct + _L - 1) // _L, replay, 0)
            cp = cpn
        pltpu.sync_copy(table_v, sum_sh.at[pl.ds(0, NUM_NODES)])
    plsc.subcore_barrier()

    def wid_init(k, _):
        w_v[pl.ds(k * _L, _L)] = lanes + (jbase + k * _L)
        return 0
    lax.fori_loop(0, _B_PER_T // _L, wid_init, 0)

    for r in range(8):
        @pl.when(r * 128 < ndirty)
        def _winners():
            nv = jnp.clip((ndirty - r * 128 + _L - 1) // _L, 0, 8)

            def mkiv(k2, _):
                pos = r * 128 + k2 * _L
                cm = dirty_v[pl.ds(pos, _L)]
                maskv = lanes < ndirty - pos
                iv = lax.shift_right_arithmetic(
                    jnp.where(maskv, cm, 0), _JBITS)
                idx_2d.at[r][pl.ds(k2 * _L, _L)] = iv
                return 0

            lax.fori_loop(0, nv, mkiv, 0)
            pltpu.sync_copy(sum_sh.at[idx_2d.at[r]], rs_v.at[r])

            def fix(k2, _):
                pos = r * 128 + k2 * _L
                cm = dirty_v[pl.ds(pos, _L)]
                wv = rs_v.at[r][pl.ds(k2 * _L, _L)]
                maskv = lanes < ndirty - pos
                jloc = jnp.bitwise_and(cm, BATCH - 1) - jbase
                plsc.store_scatter(w_v, [jloc], wv, mask=maskv)
                return 0

            lax.fori_loop(0, nv, fix, 0)

    base = wid * _B_PER_W
    woff = c * _B_PER_W
    nch = _B_PER_W // _ROWS

    def gather_cp(t, buf):
        return pltpu.async_copy(
            val_hbm.at[w_v.at[pl.ds(woff + t * _ROWS, _ROWS)]],
            rows_v.at[buf], gsem.at[buf])

    gcur = gather_cp(0, 0)
    sts = [None, None]
    for t in range(nch):
        gnxt = None
        if t + 1 < nch:
            b = (t + 1) % _NBUF
            if sts[b] is not None:
                sts[b].wait()
                sts[b] = None
            gnxt = gather_cp(t + 1, b)
        gcur.wait()
        sts[t % _NBUF] = pltpu.async_copy(
            rows_v.at[t % _NBUF],
            out_hbm.at[pl.ds(base + t * _ROWS, _ROWS)],
            wsem.at[t % _NBUF])
        gcur = gnxt
    for st in sts:
        if st is not None:
            st.wait()


def kernel(mem, idx, val):
    del mem
    run = pl.kernel(
        _body,
        out_type=jax.ShapeDtypeStruct((BATCH, MEMORY_DIM), jnp.float32),
        mesh=plsc.VectorSubcoreMesh(core_axis_name="c", subcore_axis_name="s"),
        compiler_params=pltpu.CompilerParams(needs_layout_passes=False),
        scratch_types=[
            pltpu.VMEM((NUM_NODES,), jnp.int32),
            pltpu.VMEM((8, 128), jnp.int32),
            pltpu.VMEM((8, 128), jnp.int32),
            pltpu.VMEM((_B_PER_T,), jnp.int32),
            pltpu.VMEM((_B_PER_T,), jnp.int32),
            pltpu.VMEM((_B_PER_T,), jnp.int32),
            pltpu.VMEM((_B_PER_T,), jnp.int32),
            pltpu.VMEM((_NS, _L), jnp.int32),
            pltpu.VMEM((_NBUF, _ROWS, MEMORY_DIM), jnp.float32),
            pltpu.VMEM_SHARED((_NPAD,), jnp.int32),
            pltpu.VMEM_SHARED((_NS, _B_PER_T), jnp.int32),
            pltpu.VMEM_SHARED((_NS, _L), jnp.int32),
            pltpu.SemaphoreType.DMA((_NBUF,)),
            pltpu.SemaphoreType.DMA((2,)),
            pltpu.SemaphoreType.DMA((2,)),
            pltpu.SemaphoreType.DMA((2,)),
        ],
    )
    return run(idx.reshape(BATCH // 128, 128), val)
